# baseline (speedup 1.0000x reference)
import jax
import jax.numpy as jnp
from jax.experimental import pallas as pl

N_HEADS, HEAD_DIM, ROPE_DIM, TOPK = 64, 128, 64, 1024
HEAD_CHUNK = 8
SBLK = 256
TBLK = 256
SCALE = HEAD_DIM ** -0.5
NEG = -1e9


def _fwht(x):
    d = x.shape[-1]
    shp = x.shape
    x = x.reshape(-1, d)
    h = 1
    while h < d:
        x = x.reshape(-1, d // (2 * h), 2, h)
        a = x[:, :, 0, :]
        b = x[:, :, 1, :]
        x = jnp.stack([a + b, a - b], axis=2).reshape(-1, d)
        h *= 2
    return (x * (d ** -0.5)).reshape(shp)


def _apply_rope(x, cos, sin):
    d = x.shape[-1]
    xr = x[..., : d // 2]
    xi = x[..., d // 2 :]
    c = cos.reshape(1, cos.shape[0], 1, cos.shape[1])
    s = sin.reshape(1, sin.shape[0], 1, sin.shape[1])
    return jnp.concatenate([xr * c - xi * s, xr * s + xi * c], axis=-1)


def _layer_norm(x, w, b, eps=1e-6):
    mu = jnp.mean(x, axis=-1, keepdims=True)
    var = jnp.mean((x - mu) ** 2, axis=-1, keepdims=True)
    return (x - mu) / jnp.sqrt(var + eps) * w + b


def _score_kernel(qf_ref, k_ref, w_ref, out_ref):
    i = pl.program_id(0)
    j = pl.program_id(1)

    @pl.when(j > i)
    def _skip():
        out_ref[...] = jnp.full((SBLK, TBLK), NEG, jnp.float32)

    @pl.when(j <= i)
    def _compute():
        kb = k_ref[...]
        score = None
        for c in range(N_HEADS // HEAD_CHUNK):
            ch = None
            for hh in range(HEAD_CHUNK):
                h = c * HEAD_CHUNK + hh
                qh = qf_ref[:, h * HEAD_DIM : (h + 1) * HEAD_DIM]
                sc = jax.lax.dot_general(
                    qh, kb, (((1,), (1,)), ((), ())),
                    preferred_element_type=jnp.float32)
                sc = jnp.maximum(sc, 0.0) * SCALE
                scq = sc.astype(jnp.bfloat16).astype(jnp.float32)
                wq = w_ref[:, h : h + 1].astype(jnp.bfloat16).astype(jnp.float32)
                t = wq * scq
                ch = t if ch is None else ch + t
            score = ch if score is None else score + ch

        @pl.when(j == i)
        def _mask():
            rows = jax.lax.broadcasted_iota(jnp.int32, (SBLK, TBLK), 0)
            cols = jax.lax.broadcasted_iota(jnp.int32, (SBLK, TBLK), 1)
            out_ref[...] = score + jnp.where(cols <= rows, 0.0, NEG).astype(jnp.float32)

        @pl.when(j < i)
        def _nomask():
            out_ref[...] = score


def _scores(qf, k, w, s):
    return pl.pallas_call(
        _score_kernel,
        grid=(s // SBLK, s // TBLK),
        in_specs=[
            pl.BlockSpec((SBLK, N_HEADS * HEAD_DIM), lambda i, j: (i, 0)),
            pl.BlockSpec((TBLK, HEAD_DIM), lambda i, j: (j, 0)),
            pl.BlockSpec((SBLK, N_HEADS), lambda i, j: (i, 0)),
        ],
        out_specs=pl.BlockSpec((SBLK, TBLK), lambda i, j: (i, j)),
        out_shape=jax.ShapeDtypeStruct((s, s), jnp.float32),
    )(qf, k, w)


def kernel(hidden_states, q_lora, freqs_cos, freqs_sin, wq_b, wk, k_norm_w, k_norm_b, w_proj):
    b, s, d = hidden_states.shape
    q = (q_lora @ wq_b).reshape(b, s, N_HEADS, HEAD_DIM)
    q_pe = _apply_rope(q[..., :ROPE_DIM], freqs_cos, freqs_sin)
    q = jnp.concatenate([q_pe, q[..., ROPE_DIM:]], axis=-1)
    k = hidden_states @ wk
    k = _layer_norm(k, k_norm_w, k_norm_b)
    k_pe = _apply_rope(k[..., :ROPE_DIM][:, :, None, :], freqs_cos, freqs_sin)[:, :, 0, :]
    k = jnp.concatenate([k_pe, k[..., ROPE_DIM:]], axis=-1)
    q = _fwht(q)
    k = _fwht(k)
    weights = (hidden_states @ w_proj) * (N_HEADS ** -0.5)

    qf = q.reshape(s, N_HEADS * HEAD_DIM)
    masked = _scores(qf, k[0], weights[0], s)

    topk_vals, topk_idx = jax.lax.top_k(masked[None], TOPK)
    return topk_vals, topk_idx
